# Initial kernel scaffold; baseline (speedup 1.0000x reference)
import jax
import jax.numpy as jnp
from jax.experimental import pallas as pl


def kernel(x, table):
    raise NotImplementedError("write your pallas kernel here")


# speedup vs baseline: 1.2131x; 1.2131x over previous
import functools
import math

import jax
import jax.numpy as jnp
from jax import lax
from jax.experimental import pallas as pl
from jax.experimental.pallas import tpu as pltpu
from jax.experimental.pallas import tpu_sc as plsc

D_MODEL = 768
SCALE = math.sqrt(float(D_MODEL))
LANES = 16
SLICES_PER_ROW = D_MODEL // LANES


def _make_emb_kernel(B: int, D: int, NC: int, NS: int):
    NW = NC * NS
    b_per_w = B // NW
    CH = 64
    n_chunks = b_per_w // CH
    mesh = plsc.VectorSubcoreMesh(core_axis_name="c", subcore_axis_name="s")

    @functools.partial(
        pl.kernel,
        mesh=mesh,
        out_type=jax.ShapeDtypeStruct((B, D), jnp.float32),
        scratch_types=[
            pltpu.VMEM((b_per_w,), jnp.int32),
            pltpu.VMEM((CH, D), jnp.float32),
            pltpu.SemaphoreType.DMA,
        ],
    )
    def emb(idx_hbm, table_hbm, out_hbm, idx_v, rows_v, sem):
        wid = lax.axis_index("s") * NC + lax.axis_index("c")
        base = wid * b_per_w
        pltpu.sync_copy(idx_hbm.at[pl.ds(base, b_per_w)], idx_v)

        def chunk_body(c, _):
            pltpu.async_copy(
                table_hbm.at[idx_v.at[pl.ds(c * CH, CH)]], rows_v, sem
            ).wait()

            def row_body(r, _):
                for s in range(SLICES_PER_ROW):
                    sl = pl.ds(s * LANES, LANES)
                    rows_v[r, sl] = rows_v[r, sl] * SCALE
                return 0

            lax.fori_loop(0, CH, row_body, 0)
            pltpu.sync_copy(rows_v, out_hbm.at[pl.ds(base + c * CH, CH)])
            return 0

        lax.fori_loop(0, n_chunks, chunk_body, 0)

    return emb


@jax.jit
def kernel(x, table):
    B0, S = x.shape
    V, D = table.shape
    idx = x.reshape(-1).astype(jnp.int32)
    info = plsc.get_sparse_core_info()
    emb = _make_emb_kernel(B0 * S, D, info.num_cores, info.num_subcores)
    out = emb(idx, table)
    return out.reshape(B0, S, D)


# speedup vs baseline: 1.5683x; 1.2928x over previous
import functools
import math

import jax
import jax.numpy as jnp
from jax import lax
from jax.experimental import pallas as pl
from jax.experimental.pallas import tpu as pltpu
from jax.experimental.pallas import tpu_sc as plsc

D_MODEL = 768
SCALE = math.sqrt(float(D_MODEL))
LANES = 16
SLICES_PER_ROW = D_MODEL // LANES
RING = 4


def _make_emb_kernel(B: int, D: int, NC: int, NS: int):
    NW = NC * NS
    b_per_w = B // NW
    CH = 32
    n_chunks = b_per_w // CH
    n_groups = n_chunks // RING
    mesh = plsc.VectorSubcoreMesh(core_axis_name="c", subcore_axis_name="s")

    @functools.partial(
        pl.kernel,
        mesh=mesh,
        out_type=jax.ShapeDtypeStruct((B, D), jnp.float32),
        scratch_types=[
            pltpu.VMEM((b_per_w,), jnp.int32),
            pltpu.VMEM((RING, CH, D), jnp.float32),
        ]
        + [pltpu.SemaphoreType.DMA] * (2 * RING),
    )
    def emb(idx_hbm, table_hbm, out_hbm, idx_v, rows_v, *sems):
        sem_g = sems[:RING]
        sem_o = sems[RING:]
        wid = lax.axis_index("s") * NC + lax.axis_index("c")
        base = wid * b_per_w
        pltpu.sync_copy(idx_hbm.at[pl.ds(base, b_per_w)], idx_v)

        def start_g(c, b):
            return pltpu.async_copy(
                table_hbm.at[idx_v.at[pl.ds(c * CH, CH)]], rows_v.at[b], sem_g[b]
            )

        def wait_g(c, b):
            pltpu.make_async_copy(
                table_hbm.at[idx_v.at[pl.ds(c * CH, CH)]], rows_v.at[b], sem_g[b]
            ).wait()

        def start_o(c, b):
            return pltpu.async_copy(
                rows_v.at[b], out_hbm.at[pl.ds(base + c * CH, CH)], sem_o[b]
            )

        def wait_o(c, b):
            pltpu.make_async_copy(
                rows_v.at[b], out_hbm.at[pl.ds(base + c * CH, CH)], sem_o[b]
            ).wait()

        def scale(b):
            def row_body(r, _):
                for s in range(SLICES_PER_ROW):
                    sl = pl.ds(s * LANES, LANES)
                    rows_v[b, r, sl] = rows_v[b, r, sl] * SCALE
                return 0

            lax.fori_loop(0, CH, row_body, 0)

        start_g(0, 0)
        start_g(1, 1)
        for b in range(RING):
            c = b
            wait_g(c, b)
            scale(b)
            start_o(c, b)
            if c - 2 >= 0:
                wait_o(c - 2, (c - 2) % RING)
            start_g(c + 2, (c + 2) % RING)

        def group_body(p, _):
            for b in range(RING):
                c = p * RING + b
                wait_g(c, b)
                scale(b)
                start_o(c, b)
                wait_o(c - 2, (b - 2) % RING)
                start_g(c + 2, (b + 2) % RING)
            return 0

        lax.fori_loop(1, n_groups - 1, group_body, 0)

        for b in range(RING):
            c = (n_groups - 1) * RING + b
            wait_g(c, b)
            scale(b)
            start_o(c, b)
            wait_o(c - 2, (b - 2) % RING)
            if c + 2 < n_chunks:
                start_g(c + 2, (b + 2) % RING)
        wait_o(n_chunks - 2, (n_chunks - 2) % RING)
        wait_o(n_chunks - 1, (n_chunks - 1) % RING)

    return emb


@jax.jit
def kernel(x, table):
    B0, S = x.shape
    V, D = table.shape
    idx = x.reshape(-1).astype(jnp.int32)
    info = plsc.get_sparse_core_info()
    emb = _make_emb_kernel(B0 * S, D, info.num_cores, info.num_subcores)
    out = emb(idx, table)
    return out.reshape(B0, S, D)


# speedup vs baseline: 1.5969x; 1.0182x over previous
import functools
import math

import jax
import jax.numpy as jnp
from jax import lax
from jax.experimental import pallas as pl
from jax.experimental.pallas import tpu as pltpu
from jax.experimental.pallas import tpu_sc as plsc

D_MODEL = 768
SCALE = math.sqrt(float(D_MODEL))
LANES = 16
SLICES_PER_ROW = D_MODEL // LANES
RING = 4


def _make_emb_kernel(B: int, D: int, NC: int, NS: int):
    NW = NC * NS
    b_per_w = B // NW
    CH = 32
    n_chunks = b_per_w // CH
    n_groups = n_chunks // RING
    mesh = plsc.VectorSubcoreMesh(core_axis_name="c", subcore_axis_name="s")

    @functools.partial(
        pl.kernel,
        mesh=mesh,
        out_type=jax.ShapeDtypeStruct((B, D), jnp.float32),
        scratch_types=[
            pltpu.VMEM((b_per_w,), jnp.int32),
            pltpu.VMEM((RING, CH, D), jnp.float32),
        ]
        + [pltpu.SemaphoreType.DMA] * (2 * RING),
    )
    def emb(idx_hbm, table_hbm, out_hbm, idx_v, rows_v, *sems):
        sem_g = sems[:RING]
        sem_o = sems[RING:]
        wid = lax.axis_index("s") * NC + lax.axis_index("c")
        base = wid * b_per_w
        pltpu.sync_copy(idx_hbm.at[pl.ds(base, b_per_w)], idx_v)

        def start_g(c, b):
            return pltpu.async_copy(
                table_hbm.at[idx_v.at[pl.ds(c * CH, CH)]], rows_v.at[b], sem_g[b]
            )

        def wait_g(c, b):
            pltpu.make_async_copy(
                table_hbm.at[idx_v.at[pl.ds(c * CH, CH)]], rows_v.at[b], sem_g[b]
            ).wait()

        def start_o(c, b):
            return pltpu.async_copy(
                rows_v.at[b], out_hbm.at[pl.ds(base + c * CH, CH)], sem_o[b]
            )

        def wait_o(c, b):
            pltpu.make_async_copy(
                rows_v.at[b], out_hbm.at[pl.ds(base + c * CH, CH)], sem_o[b]
            ).wait()

        def scale(b):
            @plsc.parallel_loop(0, CH)
            def row_body(r):
                for s in range(SLICES_PER_ROW):
                    sl = pl.ds(s * LANES, LANES)
                    rows_v[b, r, sl] = rows_v[b, r, sl] * SCALE

        start_g(0, 0)
        start_g(1, 1)
        for b in range(RING):
            c = b
            wait_g(c, b)
            scale(b)
            start_o(c, b)
            if c - 2 >= 0:
                wait_o(c - 2, (c - 2) % RING)
            start_g(c + 2, (c + 2) % RING)

        def group_body(p, _):
            for b in range(RING):
                c = p * RING + b
                wait_g(c, b)
                scale(b)
                start_o(c, b)
                wait_o(c - 2, (b - 2) % RING)
                start_g(c + 2, (b + 2) % RING)
            return 0

        lax.fori_loop(1, n_groups - 1, group_body, 0)

        for b in range(RING):
            c = (n_groups - 1) * RING + b
            wait_g(c, b)
            scale(b)
            start_o(c, b)
            wait_o(c - 2, (b - 2) % RING)
            if c + 2 < n_chunks:
                start_g(c + 2, (b + 2) % RING)
        wait_o(n_chunks - 2, (n_chunks - 2) % RING)
        wait_o(n_chunks - 1, (n_chunks - 1) % RING)

    return emb


@jax.jit
def kernel(x, table):
    B0, S = x.shape
    V, D = table.shape
    idx = x.reshape(-1).astype(jnp.int32)
    info = plsc.get_sparse_core_info()
    emb = _make_emb_kernel(B0 * S, D, info.num_cores, info.num_subcores)
    out = emb(idx, table)
    return out.reshape(B0, S, D)
